# Initial kernel scaffold; baseline (speedup 1.0000x reference)
import jax
import jax.numpy as jnp
from jax.experimental import pallas as pl


def kernel(anchors, objectness, box_regression):
    raise NotImplementedError("write your pallas kernel here")


# speedup vs baseline: 2.6132x; 2.6132x over previous
import math

import jax
import jax.numpy as jnp
from jax.experimental import pallas as pl
from jax.experimental.pallas import tpu as pltpu

PRE_N = 2000
PAD_N = 2048
POST_N = 1000
NMS_THRESH = 0.7
CLIP = math.log(1000.0 / 16)
RB = 256
DEG = 180.0 / math.pi
RAD = math.pi / 180.0


def _decode_row(br, anc):
    dx, dy, da = br[0:1], br[1:2], br[4:5]
    dw = jnp.minimum(br[2:3], CLIP)
    dh = jnp.minimum(br[3:4], CLIP)
    cxa, cya, wa, ha, anga = anc[0:1], anc[1:2], anc[2:3], anc[3:4], anc[4:5]
    pcx = dx * wa + cxa
    pcy = dy * ha + cya
    pw = jnp.exp(dw) * wa
    ph = jnp.exp(dh) * ha
    pa = da * DEG + anga
    return pcx, pcy, pw, ph, pa


def _decode_col(br, anc):
    dx, dy, da = br[:, 0:1], br[:, 1:2], br[:, 4:5]
    dw = jnp.minimum(br[:, 2:3], CLIP)
    dh = jnp.minimum(br[:, 3:4], CLIP)
    cxa, cya, wa, ha, anga = (
        anc[:, 0:1], anc[:, 1:2], anc[:, 2:3], anc[:, 3:4], anc[:, 4:5])
    pcx = dx * wa + cxa
    pcy = dy * ha + cya
    pw = jnp.exp(dw) * wa
    ph = jnp.exp(dh) * ha
    pa = da * DEG + anga
    return pcx, pcy, pw, ph, pa


def _envelope(pcx, pcy, pw, ph, pa):
    rad = pa * RAD
    c = jnp.abs(jnp.cos(rad))
    s = jnp.abs(jnp.sin(rad))
    ex = (pw * c + ph * s) * 0.5
    ey = (pw * s + ph * c) * 0.5
    x1 = pcx - ex
    y1 = pcy - ey
    x2 = pcx + ex
    y2 = pcy + ey
    area = (x2 - x1) * (y2 - y1)
    return x1, y1, x2, y2, area


def _nms_kernel(br_r, anc_r, br_c, anc_c, lg_r,
                prop_ref, score_ref, supp_ref, m_ref):
    pcx, pcy, pw, ph, pa = _decode_row(br_r[0], anc_r[0])
    prop_ref[0, 0:1, :] = pcx
    prop_ref[0, 1:2, :] = pcy
    prop_ref[0, 2:3, :] = pw
    prop_ref[0, 3:4, :] = ph
    prop_ref[0, 4:5, :] = pa
    prop_ref[0, 5:8, :] = jnp.zeros((3, PAD_N), jnp.float32)
    score_ref[0] = jax.nn.sigmoid(lg_r[0])

    x1, y1, x2, y2, area = _envelope(pcx, pcy, pw, ph, pa)
    x1c, y1c, x2c, y2c, areac = _envelope(*_decode_col(br_c[0], anc_c[0]))

    for rb in range(PAD_N // RB):
        sl = slice(rb * RB, (rb + 1) * RB)
        xx1 = jnp.maximum(x1c[sl], x1)
        yy1 = jnp.maximum(y1c[sl], y1)
        xx2 = jnp.minimum(x2c[sl], x2)
        yy2 = jnp.minimum(y2c[sl], y2)
        iw = jnp.maximum(xx2 - xx1, 0.0)
        ih = jnp.maximum(yy2 - yy1, 0.0)
        inter = iw * ih
        iou = inter / (areac[sl] + area - inter + 1e-9)
        m_ref[sl, :] = (iou > NMS_THRESH).astype(jnp.float32)

    iota = jax.lax.broadcasted_iota(jnp.int32, (1, PAD_N), 1)
    supp0 = (iota >= PRE_N).astype(jnp.float32)

    def body(i, supp):
        row = m_ref[pl.ds(i, 1), :]
        si = jnp.max(jnp.where(iota == i, supp, 0.0))
        sup_row = jnp.where(iota > i, row, 0.0)
        return jnp.maximum(supp, sup_row * (1.0 - si))

    supp_ref[0] = jax.lax.fori_loop(0, PRE_N, body, supp0)


def kernel(anchors, objectness, box_regression):
    n, a, h, w = objectness.shape
    obj = objectness.transpose(0, 2, 3, 1).reshape(n, -1)
    br = box_regression.reshape(n, a, 5, h, w).transpose(0, 3, 4, 1, 2)
    br = br.reshape(n, -1, 5)
    vals, idx = jax.lax.top_k(obj, PRE_N)
    br_sel = jnp.take_along_axis(br, idx[:, :, None], axis=1)
    anc_sel = jnp.take_along_axis(anchors.reshape(n, -1, 5),
                                  idx[:, :, None], axis=1)
    padn = PAD_N - PRE_N
    br_p = jnp.pad(br_sel, ((0, 0), (0, padn), (0, 0)))
    anc_p = jnp.pad(anc_sel, ((0, 0), (0, padn), (0, 0)), constant_values=1.0)
    lg_p = jnp.pad(vals, ((0, 0), (0, padn)), constant_values=-1e30)

    br_row = jnp.pad(br_p.transpose(0, 2, 1), ((0, 0), (0, 3), (0, 0)))
    anc_row = jnp.pad(anc_p.transpose(0, 2, 1), ((0, 0), (0, 3), (0, 0)))
    br_col = jnp.pad(br_p, ((0, 0), (0, 0), (0, 3)))
    anc_col = jnp.pad(anc_p, ((0, 0), (0, 0), (0, 3)))
    lg3 = lg_p.reshape(n, 1, PAD_N)

    prop, score, supp = pl.pallas_call(
        _nms_kernel,
        grid=(n,),
        in_specs=[
            pl.BlockSpec((1, 8, PAD_N), lambda i: (i, 0, 0)),
            pl.BlockSpec((1, 8, PAD_N), lambda i: (i, 0, 0)),
            pl.BlockSpec((1, PAD_N, 8), lambda i: (i, 0, 0)),
            pl.BlockSpec((1, PAD_N, 8), lambda i: (i, 0, 0)),
            pl.BlockSpec((1, 1, PAD_N), lambda i: (i, 0, 0)),
        ],
        out_specs=[
            pl.BlockSpec((1, 8, PAD_N), lambda i: (i, 0, 0)),
            pl.BlockSpec((1, 1, PAD_N), lambda i: (i, 0, 0)),
            pl.BlockSpec((1, 1, PAD_N), lambda i: (i, 0, 0)),
        ],
        out_shape=[
            jax.ShapeDtypeStruct((n, 8, PAD_N), jnp.float32),
            jax.ShapeDtypeStruct((n, 1, PAD_N), jnp.float32),
            jax.ShapeDtypeStruct((n, 1, PAD_N), jnp.float32),
        ],
        scratch_shapes=[pltpu.VMEM((PAD_N, PAD_N), jnp.float32)],
    )(br_row, anc_row, br_col, anc_col, lg3)

    props_t = prop[:, :5, :].transpose(0, 2, 1)
    scores2 = score[:, 0, :]
    suppb = supp[:, 0, :] > 0.5
    iota2 = jnp.arange(PAD_N)[None, :]
    order = jnp.sort(jnp.where(~suppb, iota2, PAD_N), axis=-1)
    keep = order[:, :POST_N]
    keep = jnp.where(keep < PAD_N, keep, 0)
    fp = jnp.take_along_axis(props_t, keep[:, :, None], axis=1)
    fs = jnp.take_along_axis(scores2, keep, axis=1)
    return fp, fs


# speedup vs baseline: 2.6989x; 1.0328x over previous
import math

import jax
import jax.numpy as jnp
from jax.experimental import pallas as pl
from jax.experimental.pallas import tpu as pltpu

PRE_N = 2000
PAD_N = 2048
POST_N = 1000
NMS_THRESH = 0.7
CLIP = math.log(1000.0 / 16)
RB = 256
DEG = 180.0 / math.pi
RAD = math.pi / 180.0


def _decode_row(br, anc):
    dx, dy, da = br[0:1], br[1:2], br[4:5]
    dw = jnp.minimum(br[2:3], CLIP)
    dh = jnp.minimum(br[3:4], CLIP)
    cxa, cya, wa, ha, anga = anc[0:1], anc[1:2], anc[2:3], anc[3:4], anc[4:5]
    pcx = dx * wa + cxa
    pcy = dy * ha + cya
    pw = jnp.exp(dw) * wa
    ph = jnp.exp(dh) * ha
    pa = da * DEG + anga
    return pcx, pcy, pw, ph, pa


def _decode_col(br, anc):
    dx, dy, da = br[:, 0:1], br[:, 1:2], br[:, 4:5]
    dw = jnp.minimum(br[:, 2:3], CLIP)
    dh = jnp.minimum(br[:, 3:4], CLIP)
    cxa, cya, wa, ha, anga = (
        anc[:, 0:1], anc[:, 1:2], anc[:, 2:3], anc[:, 3:4], anc[:, 4:5])
    pcx = dx * wa + cxa
    pcy = dy * ha + cya
    pw = jnp.exp(dw) * wa
    ph = jnp.exp(dh) * ha
    pa = da * DEG + anga
    return pcx, pcy, pw, ph, pa


def _envelope(pcx, pcy, pw, ph, pa):
    rad = pa * RAD
    c = jnp.abs(jnp.cos(rad))
    s = jnp.abs(jnp.sin(rad))
    ex = (pw * c + ph * s) * 0.5
    ey = (pw * s + ph * c) * 0.5
    x1 = pcx - ex
    y1 = pcy - ey
    x2 = pcx + ex
    y2 = pcy + ey
    area = (x2 - x1) * (y2 - y1)
    return x1, y1, x2, y2, area


def _nms_kernel(br_r, anc_r, br_c, anc_c, lg_r,
                prop_ref, score_ref, supp_ref, m_ref):
    pcx, pcy, pw, ph, pa = _decode_row(br_r[0], anc_r[0])
    prop_ref[0, 0:1, :] = pcx
    prop_ref[0, 1:2, :] = pcy
    prop_ref[0, 2:3, :] = pw
    prop_ref[0, 3:4, :] = ph
    prop_ref[0, 4:5, :] = pa
    prop_ref[0, 5:8, :] = jnp.zeros((3, PAD_N), jnp.float32)
    score_ref[0] = jax.nn.sigmoid(lg_r[0])

    x1, y1, x2, y2, area = _envelope(pcx, pcy, pw, ph, pa)
    x1c, y1c, x2c, y2c, areac = _envelope(*_decode_col(br_c[0], anc_c[0]))

    for rb in range(PAD_N // RB):
        sl = slice(rb * RB, (rb + 1) * RB)
        xx1 = jnp.maximum(x1c[sl], x1)
        yy1 = jnp.maximum(y1c[sl], y1)
        xx2 = jnp.minimum(x2c[sl], x2)
        yy2 = jnp.minimum(y2c[sl], y2)
        iw = jnp.maximum(xx2 - xx1, 0.0)
        ih = jnp.maximum(yy2 - yy1, 0.0)
        inter = iw * ih
        iou = inter / (areac[sl] + area - inter + 1e-9)
        m_ref[sl, :] = (iou > NMS_THRESH).astype(jnp.float32)

    iota = jax.lax.broadcasted_iota(jnp.int32, (1, PAD_N), 1)
    supp0 = (iota >= PRE_N).astype(jnp.float32)

    def body(i, supp):
        row = m_ref[pl.ds(i, 1), :]
        si = jnp.max(jnp.where(iota == i, supp, 0.0))
        sup_row = jnp.where(iota > i, row, 0.0)
        return jnp.maximum(supp, sup_row * (1.0 - si))

    supp_ref[0] = jax.lax.fori_loop(0, PRE_N, body, supp0)


def kernel(anchors, objectness, box_regression):
    n, a, h, w = objectness.shape
    hw = h * w
    obj = objectness.transpose(0, 2, 3, 1).reshape(n, -1)
    vals, idx = jax.lax.top_k(obj, PRE_N)
    ai = idx % a
    hwi = idx // a
    offs = ((ai[:, :, None] * 5 + jnp.arange(5)[None, None, :]) * hw
            + hwi[:, :, None])
    br_sel = jnp.take_along_axis(
        box_regression.reshape(n, -1), offs.reshape(n, -1), axis=1
    ).reshape(n, PRE_N, 5)
    anc_sel = jnp.take_along_axis(anchors.reshape(n, -1, 5),
                                  idx[:, :, None], axis=1)
    padn = PAD_N - PRE_N
    br_p = jnp.pad(br_sel, ((0, 0), (0, padn), (0, 0)))
    anc_p = jnp.pad(anc_sel, ((0, 0), (0, padn), (0, 0)), constant_values=1.0)
    lg_p = jnp.pad(vals, ((0, 0), (0, padn)), constant_values=-1e30)

    br_row = jnp.pad(br_p.transpose(0, 2, 1), ((0, 0), (0, 3), (0, 0)))
    anc_row = jnp.pad(anc_p.transpose(0, 2, 1), ((0, 0), (0, 3), (0, 0)))
    br_col = jnp.pad(br_p, ((0, 0), (0, 0), (0, 3)))
    anc_col = jnp.pad(anc_p, ((0, 0), (0, 0), (0, 3)))
    lg3 = lg_p.reshape(n, 1, PAD_N)

    prop, score, supp = pl.pallas_call(
        _nms_kernel,
        grid=(n,),
        in_specs=[
            pl.BlockSpec((1, 8, PAD_N), lambda i: (i, 0, 0)),
            pl.BlockSpec((1, 8, PAD_N), lambda i: (i, 0, 0)),
            pl.BlockSpec((1, PAD_N, 8), lambda i: (i, 0, 0)),
            pl.BlockSpec((1, PAD_N, 8), lambda i: (i, 0, 0)),
            pl.BlockSpec((1, 1, PAD_N), lambda i: (i, 0, 0)),
        ],
        out_specs=[
            pl.BlockSpec((1, 8, PAD_N), lambda i: (i, 0, 0)),
            pl.BlockSpec((1, 1, PAD_N), lambda i: (i, 0, 0)),
            pl.BlockSpec((1, 1, PAD_N), lambda i: (i, 0, 0)),
        ],
        out_shape=[
            jax.ShapeDtypeStruct((n, 8, PAD_N), jnp.float32),
            jax.ShapeDtypeStruct((n, 1, PAD_N), jnp.float32),
            jax.ShapeDtypeStruct((n, 1, PAD_N), jnp.float32),
        ],
        scratch_shapes=[pltpu.VMEM((PAD_N, PAD_N), jnp.float32)],
        compiler_params=pltpu.CompilerParams(
            dimension_semantics=("parallel",)),
    )(br_row, anc_row, br_col, anc_col, lg3)

    props_t = prop[:, :5, :].transpose(0, 2, 1)
    scores2 = score[:, 0, :]
    suppb = supp[:, 0, :] > 0.5
    iota2 = jnp.arange(PAD_N)[None, :]
    order = jnp.sort(jnp.where(~suppb, iota2, PAD_N), axis=-1)
    keep = order[:, :POST_N]
    keep = jnp.where(keep < PAD_N, keep, 0)
    fp = jnp.take_along_axis(props_t, keep[:, :, None], axis=1)
    fs = jnp.take_along_axis(scores2, keep, axis=1)
    return fp, fs


# speedup vs baseline: 13.2000x; 4.8908x over previous
import math

import jax
import jax.numpy as jnp
from jax.experimental import pallas as pl
from jax.experimental.pallas import tpu as pltpu

PRE_N = 2000
PAD_N = 2048
POST_N = 1000
NMS_THRESH = 0.7
CLIP = math.log(1000.0 / 16)
RB = 256
DEG = 180.0 / math.pi
RAD = math.pi / 180.0


def _decode_row(br, anc):
    dx, dy, da = br[0:1], br[1:2], br[4:5]
    dw = jnp.minimum(br[2:3], CLIP)
    dh = jnp.minimum(br[3:4], CLIP)
    cxa, cya, wa, ha, anga = anc[0:1], anc[1:2], anc[2:3], anc[3:4], anc[4:5]
    pcx = dx * wa + cxa
    pcy = dy * ha + cya
    pw = jnp.exp(dw) * wa
    ph = jnp.exp(dh) * ha
    pa = da * DEG + anga
    return pcx, pcy, pw, ph, pa


def _decode_col(br, anc):
    dx, dy, da = br[:, 0:1], br[:, 1:2], br[:, 4:5]
    dw = jnp.minimum(br[:, 2:3], CLIP)
    dh = jnp.minimum(br[:, 3:4], CLIP)
    cxa, cya, wa, ha, anga = (
        anc[:, 0:1], anc[:, 1:2], anc[:, 2:3], anc[:, 3:4], anc[:, 4:5])
    pcx = dx * wa + cxa
    pcy = dy * ha + cya
    pw = jnp.exp(dw) * wa
    ph = jnp.exp(dh) * ha
    pa = da * DEG + anga
    return pcx, pcy, pw, ph, pa


def _envelope(pcx, pcy, pw, ph, pa):
    rad = pa * RAD
    c = jnp.abs(jnp.cos(rad))
    s = jnp.abs(jnp.sin(rad))
    ex = (pw * c + ph * s) * 0.5
    ey = (pw * s + ph * c) * 0.5
    x1 = pcx - ex
    y1 = pcy - ey
    x2 = pcx + ex
    y2 = pcy + ey
    area = (x2 - x1) * (y2 - y1)
    return x1, y1, x2, y2, area


NCHUNK = 2048
CHUNK = 512
TOPC = 16
SB = 256


def _chunk_topk_kernel(obj_ref, vals_ref, gidx_ref):
    lane = jax.lax.broadcasted_iota(jnp.int32, (SB, CHUNK), 1)
    for sb in range(NCHUNK // SB):
        sl = slice(sb * SB, (sb + 1) * SB)
        y = obj_ref[0, sl, :]
        rowbase = (jax.lax.broadcasted_iota(jnp.int32, (SB, 1), 0)
                   + sb * SB) * CHUNK
        for r in range(TOPC):
            m = jnp.max(y, axis=1, keepdims=True)
            am = jnp.min(jnp.where(y == m, lane, CHUNK), axis=1,
                         keepdims=True)
            vals_ref[0, sl, r:r + 1] = m
            gidx_ref[0, sl, r:r + 1] = rowbase + am
            y = jnp.where(lane == am, -jnp.inf, y)


def _nms_kernel(br_r, anc_r, br_c, anc_c, lg_r,
                prop_ref, score_ref, supp_ref, m_ref):
    pcx, pcy, pw, ph, pa = _decode_row(br_r[0], anc_r[0])
    prop_ref[0, 0:1, :] = pcx
    prop_ref[0, 1:2, :] = pcy
    prop_ref[0, 2:3, :] = pw
    prop_ref[0, 3:4, :] = ph
    prop_ref[0, 4:5, :] = pa
    prop_ref[0, 5:8, :] = jnp.zeros((3, PAD_N), jnp.float32)
    score_ref[0] = jax.nn.sigmoid(lg_r[0])

    x1, y1, x2, y2, area = _envelope(pcx, pcy, pw, ph, pa)
    x1c, y1c, x2c, y2c, areac = _envelope(*_decode_col(br_c[0], anc_c[0]))

    for rb in range(PAD_N // RB):
        sl = slice(rb * RB, (rb + 1) * RB)
        xx1 = jnp.maximum(x1c[sl], x1)
        yy1 = jnp.maximum(y1c[sl], y1)
        xx2 = jnp.minimum(x2c[sl], x2)
        yy2 = jnp.minimum(y2c[sl], y2)
        iw = jnp.maximum(xx2 - xx1, 0.0)
        ih = jnp.maximum(yy2 - yy1, 0.0)
        inter = iw * ih
        iou = inter / (areac[sl] + area - inter + 1e-9)
        m_ref[sl, :] = (iou > NMS_THRESH).astype(jnp.float32)

    iota = jax.lax.broadcasted_iota(jnp.int32, (1, PAD_N), 1)
    supp0 = (iota >= PRE_N).astype(jnp.float32)

    def body(i, supp):
        row = m_ref[pl.ds(i, 1), :]
        si = jnp.max(jnp.where(iota == i, supp, 0.0))
        sup_row = jnp.where(iota > i, row, 0.0)
        return jnp.maximum(supp, sup_row * (1.0 - si))

    supp_ref[0] = jax.lax.fori_loop(0, PRE_N, body, supp0)


def kernel(anchors, objectness, box_regression):
    n, a, h, w = objectness.shape
    hw = h * w
    obj = objectness.transpose(0, 2, 3, 1).reshape(n, -1)
    cvals, cgidx = pl.pallas_call(
        _chunk_topk_kernel,
        grid=(n,),
        in_specs=[pl.BlockSpec((1, NCHUNK, CHUNK), lambda i: (i, 0, 0))],
        out_specs=[
            pl.BlockSpec((1, NCHUNK, TOPC), lambda i: (i, 0, 0)),
            pl.BlockSpec((1, NCHUNK, TOPC), lambda i: (i, 0, 0)),
        ],
        out_shape=[
            jax.ShapeDtypeStruct((n, NCHUNK, TOPC), jnp.float32),
            jax.ShapeDtypeStruct((n, NCHUNK, TOPC), jnp.int32),
        ],
        compiler_params=pltpu.CompilerParams(
            dimension_semantics=("parallel",)),
    )(obj.reshape(n, NCHUNK, CHUNK))
    vals2, pos = jax.lax.top_k(cvals.reshape(n, -1), PRE_N)
    gidx_sel = jnp.take_along_axis(cgidx.reshape(n, -1), pos, axis=1)
    ok = jnp.all(cvals[:, :, TOPC - 1] < vals2[:, PRE_N - 1][:, None])
    vals, idx = jax.lax.cond(
        ok,
        lambda: (vals2, gidx_sel),
        lambda: tuple(jax.lax.top_k(obj, PRE_N)),
    )
    ai = idx % a
    hwi = idx // a
    offs = ((ai[:, :, None] * 5 + jnp.arange(5)[None, None, :]) * hw
            + hwi[:, :, None])
    br_sel = jnp.take_along_axis(
        box_regression.reshape(n, -1), offs.reshape(n, -1), axis=1
    ).reshape(n, PRE_N, 5)
    anc_sel = jnp.take_along_axis(anchors.reshape(n, -1, 5),
                                  idx[:, :, None], axis=1)
    padn = PAD_N - PRE_N
    br_p = jnp.pad(br_sel, ((0, 0), (0, padn), (0, 0)))
    anc_p = jnp.pad(anc_sel, ((0, 0), (0, padn), (0, 0)), constant_values=1.0)
    lg_p = jnp.pad(vals, ((0, 0), (0, padn)), constant_values=-1e30)

    br_row = jnp.pad(br_p.transpose(0, 2, 1), ((0, 0), (0, 3), (0, 0)))
    anc_row = jnp.pad(anc_p.transpose(0, 2, 1), ((0, 0), (0, 3), (0, 0)))
    br_col = jnp.pad(br_p, ((0, 0), (0, 0), (0, 3)))
    anc_col = jnp.pad(anc_p, ((0, 0), (0, 0), (0, 3)))
    lg3 = lg_p.reshape(n, 1, PAD_N)

    prop, score, supp = pl.pallas_call(
        _nms_kernel,
        grid=(n,),
        in_specs=[
            pl.BlockSpec((1, 8, PAD_N), lambda i: (i, 0, 0)),
            pl.BlockSpec((1, 8, PAD_N), lambda i: (i, 0, 0)),
            pl.BlockSpec((1, PAD_N, 8), lambda i: (i, 0, 0)),
            pl.BlockSpec((1, PAD_N, 8), lambda i: (i, 0, 0)),
            pl.BlockSpec((1, 1, PAD_N), lambda i: (i, 0, 0)),
        ],
        out_specs=[
            pl.BlockSpec((1, 8, PAD_N), lambda i: (i, 0, 0)),
            pl.BlockSpec((1, 1, PAD_N), lambda i: (i, 0, 0)),
            pl.BlockSpec((1, 1, PAD_N), lambda i: (i, 0, 0)),
        ],
        out_shape=[
            jax.ShapeDtypeStruct((n, 8, PAD_N), jnp.float32),
            jax.ShapeDtypeStruct((n, 1, PAD_N), jnp.float32),
            jax.ShapeDtypeStruct((n, 1, PAD_N), jnp.float32),
        ],
        scratch_shapes=[pltpu.VMEM((PAD_N, PAD_N), jnp.float32)],
        compiler_params=pltpu.CompilerParams(
            dimension_semantics=("parallel",)),
    )(br_row, anc_row, br_col, anc_col, lg3)

    props_t = prop[:, :5, :].transpose(0, 2, 1)
    scores2 = score[:, 0, :]
    suppb = supp[:, 0, :] > 0.5
    iota2 = jnp.arange(PAD_N)[None, :]
    order = jnp.sort(jnp.where(~suppb, iota2, PAD_N), axis=-1)
    keep = order[:, :POST_N]
    keep = jnp.where(keep < PAD_N, keep, 0)
    fp = jnp.take_along_axis(props_t, keep[:, :, None], axis=1)
    fs = jnp.take_along_axis(scores2, keep, axis=1)
    return fp, fs
